# baseline (speedup 1.0000x reference)
import functools
import math

import jax
import jax.numpy as jnp
from jax import lax
from jax.experimental import pallas as pl
from jax.experimental.pallas import tpu as pltpu
from jax.experimental.pallas import tpu_sc as plsc

B = 16384
D = 128
NC = 2
NS = 16
NW = NC * NS
BPW = B // NW
CH = 128
NCHUNK = BPW // CH
SCALE = math.sqrt(D)

_mesh = plsc.VectorSubcoreMesh(core_axis_name="c", subcore_axis_name="s")


@functools.partial(
    pl.kernel,
    mesh=_mesh,
    out_type=jax.ShapeDtypeStruct((B, D), jnp.float32),
    scratch_types=[
        pltpu.VMEM((BPW,), jnp.int32),
        pltpu.VMEM((CH, D), jnp.float32),
        pltpu.VMEM((CH, D), jnp.float32),
        pltpu.VMEM((CH, D), jnp.float32),
        pltpu.VMEM((CH, D), jnp.float32),
        pltpu.SemaphoreType.DMA,
        pltpu.SemaphoreType.DMA,
        pltpu.SemaphoreType.DMA,
        pltpu.SemaphoreType.DMA,
        pltpu.SemaphoreType.DMA,
        pltpu.SemaphoreType.DMA,
    ],
)
def _distmult_sc(node_hbm, idx_hbm, table_hbm, out_hbm,
                 idx_v, rows0, rows1, node0, node1,
                 sg0, sg1, sn0, sn1, so0, so1):
    wid = lax.axis_index("s") * NC + lax.axis_index("c")
    base = wid * BPW
    pltpu.sync_copy(idx_hbm.at[pl.ds(base, BPW)], idx_v)
    rows = (rows0, rows1)
    node = (node0, node1)
    sg = (sg0, sg1)
    sn = (sn0, sn1)
    so = (so0, so1)

    def start(c):
        b = c % 2
        g = pltpu.async_copy(table_hbm.at[idx_v.at[pl.ds(c * CH, CH)]],
                             rows[b], sg[b])
        n = pltpu.async_copy(node_hbm.at[pl.ds(base + c * CH, CH)],
                             node[b], sn[b])
        return g, n

    inflight = [None] * NCHUNK
    outflight = [None] * NCHUNK
    inflight[0] = start(0)
    for c in range(NCHUNK):
        b = c % 2
        if c + 1 < NCHUNK:
            if c - 1 >= 0:
                outflight[c - 1].wait()
            inflight[c + 1] = start(c + 1)
        g, n = inflight[c]
        g.wait()
        n.wait()

        @plsc.parallel_loop(0, CH, unroll=2)
        def _row_body(r, b=b):
            for i in range(D // 16):
                sl = pl.ds(i * 16, 16)
                rows[b][r, sl] = rows[b][r, sl] * (node[b][r, sl] * SCALE)
        outflight[c] = pltpu.async_copy(
            rows[b], out_hbm.at[pl.ds(base + c * CH, CH)], so[b])
    outflight[NCHUNK - 2].wait()
    outflight[NCHUNK - 1].wait()


def kernel(node_emb, relation, rela_emb_weight):
    idx = relation.astype(jnp.int32)
    return _distmult_sc(node_emb, idx, rela_emb_weight)


# speedup vs baseline: 1.0422x; 1.0422x over previous
import functools
import math

import jax
import jax.numpy as jnp
from jax import lax
from jax.experimental import pallas as pl
from jax.experimental.pallas import tpu as pltpu
from jax.experimental.pallas import tpu_sc as plsc

B = 16384
D = 128
NC = 2
NS = 16
NW = NC * NS
BPW = B // NW
CH = 128
NCHUNK = BPW // CH
SCALE = math.sqrt(D)

_mesh = plsc.VectorSubcoreMesh(core_axis_name="c", subcore_axis_name="s")


@functools.partial(
    pl.kernel,
    mesh=_mesh,
    out_type=jax.ShapeDtypeStruct((B, D), jnp.float32),
    scratch_types=[
        pltpu.VMEM((BPW,), jnp.int32),
        pltpu.VMEM((BPW, D), jnp.float32),
        pltpu.VMEM((CH, D), jnp.float32),
        pltpu.VMEM((CH, D), jnp.float32),
        pltpu.VMEM((CH, D), jnp.float32),
        pltpu.SemaphoreType.DMA,
        pltpu.SemaphoreType.DMA,
        pltpu.SemaphoreType.DMA,
        pltpu.SemaphoreType.DMA,
        pltpu.SemaphoreType.DMA,
        pltpu.SemaphoreType.DMA,
        pltpu.SemaphoreType.DMA,
    ],
)
def _distmult_sc(node_hbm, idx_hbm, table_hbm, out_hbm,
                 idx_v, node_v, rows0, rows1, rows2,
                 sg0, sg1, sg2, sn, so0, so1, so2):
    wid = lax.axis_index("s") * NC + lax.axis_index("c")
    base = wid * BPW
    pltpu.sync_copy(idx_hbm.at[pl.ds(base, BPW)], idx_v)
    cn = pltpu.async_copy(node_hbm.at[pl.ds(base, BPW)], node_v, sn)
    rows = (rows0, rows1, rows2)
    sg = (sg0, sg1, sg2)
    so = (so0, so1, so2)

    def start(c):
        b = c % 3
        return pltpu.async_copy(table_hbm.at[idx_v.at[pl.ds(c * CH, CH)]],
                                rows[b], sg[b])

    inflight = [None] * NCHUNK
    outflight = [None] * NCHUNK
    inflight[0] = start(0)
    inflight[1] = start(1)
    node_waited = False
    for c in range(NCHUNK):
        b = c % 3
        inflight[c].wait()
        if not node_waited:
            cn.wait()
            node_waited = True

        def row_body(r, _, b=b, c=c):
            for i in range(D // 16):
                sl = pl.ds(i * 16, 16)
                rows[b][r, sl] = rows[b][r, sl] * (node_v[c * CH + r, sl] * SCALE)
            return 0

        lax.fori_loop(0, CH, row_body, 0)
        outflight[c] = pltpu.async_copy(
            rows[b], out_hbm.at[pl.ds(base + c * CH, CH)], so[b])
        if c + 2 < NCHUNK:
            if c - 1 >= 0:
                outflight[c - 1].wait()
            inflight[c + 2] = start(c + 2)
    outflight[NCHUNK - 2].wait()
    outflight[NCHUNK - 1].wait()


def kernel(node_emb, relation, rela_emb_weight):
    idx = relation.astype(jnp.int32)
    return _distmult_sc(node_emb, idx, rela_emb_weight)


# speedup vs baseline: 1.0577x; 1.0149x over previous
import functools
import math

import jax
import jax.numpy as jnp
from jax import lax
from jax.experimental import pallas as pl
from jax.experimental.pallas import tpu as pltpu
from jax.experimental.pallas import tpu_sc as plsc

B = 16384
D = 128
NC = 2
NS = 16
NW = NC * NS
BPW = B // NW
CH = 64
NCHUNK = BPW // CH
NBUF = 4
SCALE = math.sqrt(D)

_mesh = plsc.VectorSubcoreMesh(core_axis_name="c", subcore_axis_name="s")


@functools.partial(
    pl.kernel,
    mesh=_mesh,
    out_type=jax.ShapeDtypeStruct((B, D), jnp.float32),
    scratch_types=[
        pltpu.VMEM((BPW,), jnp.int32),
        pltpu.VMEM((BPW, D), jnp.float32),
        pltpu.VMEM((CH, D), jnp.float32),
        pltpu.VMEM((CH, D), jnp.float32),
        pltpu.VMEM((CH, D), jnp.float32),
        pltpu.VMEM((CH, D), jnp.float32),
        pltpu.SemaphoreType.DMA,
        pltpu.SemaphoreType.DMA,
        pltpu.SemaphoreType.DMA,
        pltpu.SemaphoreType.DMA,
        pltpu.SemaphoreType.DMA,
        pltpu.SemaphoreType.DMA,
        pltpu.SemaphoreType.DMA,
        pltpu.SemaphoreType.DMA,
        pltpu.SemaphoreType.DMA,
    ],
)
def _distmult_sc(node_hbm, idx_hbm, table_hbm, out_hbm,
                 idx_v, node_v, rows0, rows1, rows2, rows3,
                 sg0, sg1, sg2, sg3, sn, so0, so1, so2, so3):
    wid = lax.axis_index("s") * NC + lax.axis_index("c")
    base = wid * BPW
    pltpu.sync_copy(idx_hbm.at[pl.ds(base, BPW)], idx_v)
    cn = pltpu.async_copy(node_hbm.at[pl.ds(base, BPW)], node_v, sn)
    rows = (rows0, rows1, rows2, rows3)
    sg = (sg0, sg1, sg2, sg3)
    so = (so0, so1, so2, so3)

    def start(c):
        b = c % NBUF
        return pltpu.async_copy(table_hbm.at[idx_v.at[pl.ds(c * CH, CH)]],
                                rows[b], sg[b])

    inflight = [None] * NCHUNK
    outflight = [None] * NCHUNK
    out_waited = [False] * NCHUNK
    for c in range(NBUF - 1):
        inflight[c] = start(c)
    node_waited = False
    for c in range(NCHUNK):
        b = c % NBUF
        inflight[c].wait()
        if not node_waited:
            cn.wait()
            node_waited = True

        def row_body(r, _, b=b, c=c):
            for i in range(D // 16):
                sl = pl.ds(i * 16, 16)
                rows[b][r, sl] = rows[b][r, sl] * (node_v[c * CH + r, sl] * SCALE)
            return 0

        lax.fori_loop(0, CH, row_body, 0)
        outflight[c] = pltpu.async_copy(
            rows[b], out_hbm.at[pl.ds(base + c * CH, CH)], so[b])
        nxt = c + NBUF - 1
        if nxt < NCHUNK:
            prev = nxt - NBUF
            if prev >= 0:
                outflight[prev].wait()
                out_waited[prev] = True
            inflight[nxt] = start(nxt)
    for c in range(NCHUNK):
        if not out_waited[c]:
            outflight[c].wait()


def kernel(node_emb, relation, rela_emb_weight):
    idx = relation.astype(jnp.int32)
    return _distmult_sc(node_emb, idx, rela_emb_weight)
